# baseline (speedup 1.0000x reference)
import functools

import jax
import jax.numpy as jnp
from jax import lax
from jax.experimental import pallas as pl
from jax.experimental.pallas import tpu as pltpu
from jax.experimental.pallas import tpu_sc as plsc

_LANES = 16
_CHUNK = 128
_EMBED = 64


def _make_sc_kernel(batch, embed_dim, num_workers, nc, ns):
    b_per_w = batch // num_workers
    n_chunks = b_per_w // _CHUNK
    n_groups = b_per_w // _LANES
    mesh = plsc.VectorSubcoreMesh(core_axis_name="c", subcore_axis_name="s")

    @functools.partial(
        pl.kernel,
        mesh=mesh,
        out_type=jax.ShapeDtypeStruct((batch,), jnp.float32),
        compiler_params=pltpu.CompilerParams(
            needs_layout_passes=False, use_tc_tiling_on_sc=False),
        scratch_types=[
            pltpu.VMEM((n_chunks, _CHUNK), jnp.int32),
            pltpu.VMEM((n_chunks, _CHUNK), jnp.int32),
            pltpu.VMEM((b_per_w, embed_dim), jnp.float32),
            pltpu.VMEM((b_per_w, embed_dim), jnp.float32),
            pltpu.VMEM((b_per_w,), jnp.float32),
            pltpu.VMEM((b_per_w,), jnp.float32),
            pltpu.VMEM((_LANES,), jnp.float32),
            pltpu.VMEM((b_per_w,), jnp.float32),
            pltpu.SemaphoreType.DMA,
        ],
    )
    def k(uids_r, mids_r, ue_r, me_r, ub_r, mb_r, gb_r, out_r,
          idx_u, idx_m, rows_u, rows_m, bu_v, bm_v, gb_v, out_v, sem):
        wid = lax.axis_index("c") * ns + lax.axis_index("s")

        pltpu.sync_copy(uids_r.at[pl.ds(wid * n_chunks, n_chunks)], idx_u)
        pltpu.sync_copy(mids_r.at[pl.ds(wid * n_chunks, n_chunks)], idx_m)
        pltpu.sync_copy(gb_r, gb_v)

        descs = []
        for j in range(n_chunks):
            sl = pl.ds(j * _CHUNK, _CHUNK)
            descs.append(pltpu.async_copy(ue_r.at[idx_u.at[j]], rows_u.at[sl], sem))
            descs.append(pltpu.async_copy(me_r.at[idx_m.at[j]], rows_m.at[sl], sem))
            descs.append(pltpu.async_copy(ub_r.at[idx_u.at[j]], bu_v.at[sl], sem))
            descs.append(pltpu.async_copy(mb_r.at[idx_m.at[j]], bm_v.at[sl], sem))
        for d in descs:
            d.wait()

        gb = gb_v[...]

        def group(g, carry):
            rid = lax.iota(jnp.int32, _LANES) + g * _LANES
            acc = (plsc.load_gather(bu_v, [rid])
                   + plsc.load_gather(bm_v, [rid]) + gb)
            for d in range(embed_dim):
                col = jnp.full((_LANES,), d, jnp.int32)
                acc = acc + (plsc.load_gather(rows_u, [rid, col])
                             * plsc.load_gather(rows_m, [rid, col]))
            plsc.store_scatter(out_v, [rid], acc)
            return carry

        lax.fori_loop(0, n_groups, group, 0)
        pltpu.sync_copy(out_v, out_r.at[pl.ds(wid * b_per_w, b_per_w)])

    return k


def kernel(user_ids, movie_ids, user_embedding, movie_embedding,
           user_bias, movie_bias, global_bias):
    batch = user_ids.shape[0]
    embed_dim = user_embedding.shape[1]
    info = plsc.get_sparse_core_info()
    nc, ns = info.num_cores, info.num_subcores
    num_workers = nc * ns

    k = _make_sc_kernel(batch, embed_dim, num_workers, nc, ns)
    uids2 = user_ids.astype(jnp.int32).reshape(batch // _CHUNK, _CHUNK)
    mids2 = movie_ids.astype(jnp.int32).reshape(batch // _CHUNK, _CHUNK)
    gb16 = jnp.broadcast_to(jnp.reshape(global_bias, (1,)),
                            (_LANES,)).astype(jnp.float32)
    return k(uids2, mids2, user_embedding, movie_embedding,
             user_bias.reshape(-1), movie_bias.reshape(-1), gb16)
